# Initial kernel scaffold; baseline (speedup 1.0000x reference)
import jax
import jax.numpy as jnp
from jax.experimental import pallas as pl


def kernel(x, edge_index, W1l, b1l, W1r, bn_gamma, bn_beta, W2l, b2l, W2r):
    raise NotImplementedError("write your pallas kernel here")


# speedup vs baseline: 2.5881x; 2.5881x over previous
import functools

import jax
import jax.numpy as jnp
from jax import lax
from jax.experimental import pallas as pl
from jax.experimental.pallas import tpu as pltpu
from jax.experimental.pallas import tpu_sc as plsc

N_NODES = 10000
N_EDGES = 160000
D_FEAT = 256
H = 128
NC, NS = 2, 16
C = 128
PAD_E = 163840
EPT = PAD_E // NS
NCH = EPT // C
CCH = PAD_E // (NC * NS * C)
PAD_N = 10240
RPT = PAD_N // NS
RW = 128
NWB = RPT // RW
RB = 2000
EPS = 1e-5


def _sc_agg_kernel(h_flat, bsrc_hbm, dst_hbm, zeros_hbm, sum_flat,
                   acc, src_c, dst_c, rows_v, sem):
    c = lax.axis_index("c")
    s = lax.axis_index("s")

    pltpu.sync_copy(zeros_hbm, rows_v)
    base = s * RPT
    for k in range(NWB):
        pltpu.sync_copy(rows_v, acc.at[pl.ds(base + k * RW, RW)])
    plsc.subcore_barrier()

    def chunk(j, carry):
        e = (s * NCH + j) * C
        pltpu.sync_copy(bsrc_hbm.at[pl.ds(c * PAD_E + e, C)], src_c)
        pltpu.sync_copy(dst_hbm.at[pl.ds(e, C)], dst_c)
        pltpu.async_copy(h_flat.at[src_c], rows_v, sem).wait()
        pltpu.sync_copy(rows_v, acc.at[dst_c], add=True)
        return carry

    lax.fori_loop(0, NCH, chunk, 0)
    plsc.subcore_barrier()

    for k in range(NWB):
        b = base + k * RW
        pltpu.sync_copy(acc.at[pl.ds(b, RW)],
                        sum_flat.at[pl.ds(c * PAD_N + b, RW)])


_sc_agg = pl.kernel(
    _sc_agg_kernel,
    out_type=jax.ShapeDtypeStruct((NC * PAD_N, H), jnp.float32),
    mesh=plsc.VectorSubcoreMesh(
        core_axis_name="c", subcore_axis_name="s", num_cores=NC, num_subcores=NS
    ),
    scratch_types=[
        pltpu.VMEM_SHARED((PAD_N, H), jnp.float32),
        pltpu.VMEM((C,), jnp.int32),
        pltpu.VMEM((C,), jnp.int32),
        pltpu.VMEM((C, H), jnp.float32),
        pltpu.SemaphoreType.DMA,
    ],
)


def _sc_cnt_kernel(dst_hbm, ones_hbm, zeros_hbm, cnt_out, cacc, dst_c, ones_v):
    c = lax.axis_index("c")
    s = lax.axis_index("s")
    w = c * NS + s
    base = s * RPT

    pltpu.sync_copy(zeros_hbm, ones_v)
    for k in range(NWB):
        pltpu.sync_copy(ones_v, cacc.at[pl.ds(base + k * RW, RW)])
    pltpu.sync_copy(ones_hbm, ones_v)
    plsc.subcore_barrier()

    def chunk(j, carry):
        pltpu.sync_copy(dst_hbm.at[pl.ds((w * CCH + j) * C, C)], dst_c)
        pltpu.sync_copy(ones_v, cacc.at[dst_c], add=True)
        return carry

    lax.fori_loop(0, CCH, chunk, 0)
    plsc.subcore_barrier()

    for k in range(NWB):
        b = base + k * RW
        pltpu.sync_copy(cacc.at[pl.ds(b, RW)],
                        cnt_out.at[pl.ds(c * PAD_N + b, RW)])


_sc_cnt = pl.kernel(
    _sc_cnt_kernel,
    out_type=jax.ShapeDtypeStruct((NC * PAD_N, H), jnp.float32),
    mesh=plsc.VectorSubcoreMesh(
        core_axis_name="c", subcore_axis_name="s", num_cores=NC, num_subcores=NS
    ),
    scratch_types=[
        pltpu.VMEM_SHARED((PAD_N, H), jnp.float32),
        pltpu.VMEM((C,), jnp.int32),
        pltpu.VMEM((C, H), jnp.float32),
    ],
)


def _dotT(a, w):
    return lax.dot_general(a, w, (((1,), (1,)), ((), ())),
                           preferred_element_type=jnp.float32)


def _lin1_kernel(x_ref, wl_ref, bl_ref, wr_ref, h_ref, xr_ref):
    x = x_ref[...]
    h = _dotT(x, wl_ref[...]) + bl_ref[...]
    h_ref[0] = h[:, :H]
    h_ref[1] = h[:, H:]
    xr_ref[...] = _dotT(x, wr_ref[...])


def _lin1(x, wl, bl, wr):
    return pl.pallas_call(
        _lin1_kernel,
        grid=(N_NODES // RB,),
        in_specs=[
            pl.BlockSpec((RB, D_FEAT), lambda i: (i, 0)),
            pl.BlockSpec((D_FEAT, D_FEAT), lambda i: (0, 0)),
            pl.BlockSpec((1, D_FEAT), lambda i: (0, 0)),
            pl.BlockSpec((D_FEAT, D_FEAT), lambda i: (0, 0)),
        ],
        out_specs=[
            pl.BlockSpec((NC, RB, H), lambda i: (0, i, 0)),
            pl.BlockSpec((RB, D_FEAT), lambda i: (i, 0)),
        ],
        out_shape=[
            jax.ShapeDtypeStruct((NC, N_NODES, H), jnp.float32),
            jax.ShapeDtypeStruct((N_NODES, D_FEAT), jnp.float32),
        ],
    )(x, wl, bl, wr)


def _mid_kernel(sum_ref, cnt_ref, xr_ref, t_ref, st_ref):
    i = pl.program_id(0)
    cnt = cnt_ref[0, :, 0:1] + cnt_ref[1, :, 0:1]
    inv = 1.0 / jnp.maximum(cnt, 1.0)
    agg = jnp.concatenate([sum_ref[0], sum_ref[1]], axis=1) * inv
    t = agg + xr_ref[...]
    t_ref[...] = t

    @pl.when(i == 0)
    def _():
        st_ref[...] = jnp.zeros_like(st_ref)

    st_ref[0:1, :] = st_ref[0:1, :] + jnp.sum(t, axis=0, keepdims=True)
    st_ref[1:2, :] = st_ref[1:2, :] + jnp.sum(t * t, axis=0, keepdims=True)


def _mid(sum_stk, cnt, xr):
    return pl.pallas_call(
        _mid_kernel,
        grid=(N_NODES // RB,),
        in_specs=[
            pl.BlockSpec((NC, RB, H), lambda i: (0, i, 0)),
            pl.BlockSpec((NC, RB, H), lambda i: (0, i, 0)),
            pl.BlockSpec((RB, D_FEAT), lambda i: (i, 0)),
        ],
        out_specs=[
            pl.BlockSpec((RB, D_FEAT), lambda i: (i, 0)),
            pl.BlockSpec((2, D_FEAT), lambda i: (0, 0)),
        ],
        out_shape=[
            jax.ShapeDtypeStruct((N_NODES, D_FEAT), jnp.float32),
            jax.ShapeDtypeStruct((2, D_FEAT), jnp.float32),
        ],
    )(sum_stk, cnt, xr)


def _bn_lin2_kernel(t_ref, st_ref, g_ref, be_ref, wl_ref, bl_ref, wr_ref,
                    h2_ref, x2r_ref):
    n = float(N_NODES)
    mean = st_ref[0:1, :] / n
    var = st_ref[1:2, :] / n - mean * mean
    u = (t_ref[...] - mean) * lax.rsqrt(var + EPS) * g_ref[...] + be_ref[...]
    u = jnp.maximum(u, 0.0)
    h2 = _dotT(u, wl_ref[...]) + bl_ref[...]
    h2_ref[0] = h2[:, :H]
    h2_ref[1] = h2[:, H:]
    x2r_ref[...] = _dotT(u, wr_ref[...])


def _bn_lin2(t, st, gamma, beta, wl, bl, wr):
    return pl.pallas_call(
        _bn_lin2_kernel,
        grid=(N_NODES // RB,),
        in_specs=[
            pl.BlockSpec((RB, D_FEAT), lambda i: (i, 0)),
            pl.BlockSpec((2, D_FEAT), lambda i: (0, 0)),
            pl.BlockSpec((1, D_FEAT), lambda i: (0, 0)),
            pl.BlockSpec((1, D_FEAT), lambda i: (0, 0)),
            pl.BlockSpec((D_FEAT, D_FEAT), lambda i: (0, 0)),
            pl.BlockSpec((1, D_FEAT), lambda i: (0, 0)),
            pl.BlockSpec((D_FEAT, D_FEAT), lambda i: (0, 0)),
        ],
        out_specs=[
            pl.BlockSpec((NC, RB, H), lambda i: (0, i, 0)),
            pl.BlockSpec((RB, D_FEAT), lambda i: (i, 0)),
        ],
        out_shape=[
            jax.ShapeDtypeStruct((NC, N_NODES, H), jnp.float32),
            jax.ShapeDtypeStruct((N_NODES, D_FEAT), jnp.float32),
        ],
    )(t, st, gamma, beta, wl, bl, wr)


def _out_kernel(sum_ref, cnt_ref, x2r_ref, o_ref):
    cnt = cnt_ref[0, :, 0:1] + cnt_ref[1, :, 0:1]
    inv = 1.0 / jnp.maximum(cnt, 1.0)
    agg = jnp.concatenate([sum_ref[0], sum_ref[1]], axis=1) * inv
    o_ref[...] = agg + x2r_ref[...]


def _out(sum_stk, cnt, x2r):
    return pl.pallas_call(
        _out_kernel,
        grid=(N_NODES // RB,),
        in_specs=[
            pl.BlockSpec((NC, RB, H), lambda i: (0, i, 0)),
            pl.BlockSpec((NC, RB, H), lambda i: (0, i, 0)),
            pl.BlockSpec((RB, D_FEAT), lambda i: (i, 0)),
        ],
        out_specs=pl.BlockSpec((RB, D_FEAT), lambda i: (i, 0)),
        out_shape=jax.ShapeDtypeStruct((N_NODES, D_FEAT), jnp.float32),
    )(sum_stk, cnt, x2r)


@jax.jit
def kernel(x, edge_index, W1l, b1l, W1r, bn_gamma, bn_beta, W2l, b2l, W2r):
    npad = PAD_E - N_EDGES
    src = jnp.concatenate(
        [edge_index[0].astype(jnp.int32), jnp.zeros((npad,), jnp.int32)])
    dst = jnp.concatenate(
        [edge_index[1].astype(jnp.int32),
         jnp.full((npad,), N_NODES, jnp.int32)])
    bsrc = jnp.concatenate([src, src + N_NODES])
    ones_in = jnp.ones((C, H), jnp.float32)
    zeros_in = jnp.zeros((C, H), jnp.float32)

    cnt = _sc_cnt(dst, ones_in, zeros_in).reshape(NC, PAD_N, H)
    h_stk, xr = _lin1(x, W1l, b1l.reshape(1, -1), W1r)
    sum_stk = _sc_agg(h_stk.reshape(NC * N_NODES, H), bsrc, dst, zeros_in)
    t, st = _mid(sum_stk.reshape(NC, PAD_N, H), cnt, xr)
    h2_stk, x2r = _bn_lin2(t, st, bn_gamma.reshape(1, -1), bn_beta.reshape(1, -1),
                           W2l, b2l.reshape(1, -1), W2r)
    sum2_stk = _sc_agg(h2_stk.reshape(NC * N_NODES, H), bsrc, dst, zeros_in)
    return _out(sum2_stk.reshape(NC, PAD_N, H), cnt, x2r)


# speedup vs baseline: 3.3291x; 1.2863x over previous
import functools

import jax
import jax.numpy as jnp
from jax import lax
from jax.experimental import pallas as pl
from jax.experimental.pallas import tpu as pltpu
from jax.experimental.pallas import tpu_sc as plsc

N_NODES = 10000
N_EDGES = 160000
D_FEAT = 256
H = 128
NC, NS = 2, 16
C = 128
PAD_E = 163840
EPT = PAD_E // NS
NCH = EPT // C
CCH = PAD_E // (NC * NS * C)
PAD_N = 10240
RPT = PAD_N // NS
RW = 128
NWB = RPT // RW
RB = 2000
EPS = 1e-5


def _sc_agg_kernel(h_flat, bsrc_hbm, dst_hbm, zeros_hbm, sum_flat,
                   acc, sa0, sa1, da0, da1, rv0, rv1, sem0, sem1):
    c = lax.axis_index("c")
    s = lax.axis_index("s")
    base = s * RPT

    pltpu.sync_copy(zeros_hbm, rv0)
    for k in range(NWB):
        pltpu.sync_copy(rv0, acc.at[pl.ds(base + k * RW, RW)])
    plsc.subcore_barrier()

    def stage(j, sa, da):
        e = (s * NCH + j) * C
        pltpu.sync_copy(bsrc_hbm.at[pl.ds(c * PAD_E + e, C)], sa)
        pltpu.sync_copy(dst_hbm.at[pl.ds(e, C)], da)

    stage(0, sa0, da0)
    pltpu.async_copy(h_flat.at[sa0], rv0, sem0)

    def pair(p, carry):
        stage(2 * p + 1, sa1, da1)
        pltpu.async_copy(h_flat.at[sa1], rv1, sem1)
        pltpu.make_async_copy(h_flat.at[sa0], rv0, sem0).wait()
        pltpu.sync_copy(rv0, acc.at[da0], add=True)

        @pl.when(p < NCH // 2 - 1)
        def _():
            stage(2 * p + 2, sa0, da0)
            pltpu.async_copy(h_flat.at[sa0], rv0, sem0)

        pltpu.make_async_copy(h_flat.at[sa1], rv1, sem1).wait()
        pltpu.sync_copy(rv1, acc.at[da1], add=True)
        return carry

    lax.fori_loop(0, NCH // 2, pair, 0)
    plsc.subcore_barrier()

    for k in range(NWB):
        b = base + k * RW
        pltpu.sync_copy(acc.at[pl.ds(b, RW)],
                        sum_flat.at[pl.ds(c * PAD_N + b, RW)])


_sc_agg = pl.kernel(
    _sc_agg_kernel,
    out_type=jax.ShapeDtypeStruct((NC * PAD_N, H), jnp.float32),
    mesh=plsc.VectorSubcoreMesh(
        core_axis_name="c", subcore_axis_name="s", num_cores=NC, num_subcores=NS
    ),
    scratch_types=[
        pltpu.VMEM_SHARED((PAD_N, H), jnp.float32),
        pltpu.VMEM((C,), jnp.int32),
        pltpu.VMEM((C,), jnp.int32),
        pltpu.VMEM((C,), jnp.int32),
        pltpu.VMEM((C,), jnp.int32),
        pltpu.VMEM((C, H), jnp.float32),
        pltpu.VMEM((C, H), jnp.float32),
        pltpu.SemaphoreType.DMA,
        pltpu.SemaphoreType.DMA,
    ],
)


def _sc_cnt_kernel(dst_hbm, ones_hbm, zeros_hbm, cnt_out, cacc, dst_c, ones_v):
    c = lax.axis_index("c")
    s = lax.axis_index("s")
    w = c * NS + s
    base = s * RPT

    pltpu.sync_copy(zeros_hbm, ones_v)
    for k in range(NWB):
        pltpu.sync_copy(ones_v, cacc.at[pl.ds(base + k * RW, RW)])
    pltpu.sync_copy(ones_hbm, ones_v)
    plsc.subcore_barrier()

    def chunk(j, carry):
        pltpu.sync_copy(dst_hbm.at[pl.ds((w * CCH + j) * C, C)], dst_c)
        pltpu.sync_copy(ones_v, cacc.at[dst_c], add=True)
        return carry

    lax.fori_loop(0, CCH, chunk, 0)
    plsc.subcore_barrier()

    for k in range(NWB):
        b = base + k * RW
        pltpu.sync_copy(cacc.at[pl.ds(b, RW)],
                        cnt_out.at[pl.ds(c * PAD_N + b, RW)])


_sc_cnt = pl.kernel(
    _sc_cnt_kernel,
    out_type=jax.ShapeDtypeStruct((NC * PAD_N, H), jnp.float32),
    mesh=plsc.VectorSubcoreMesh(
        core_axis_name="c", subcore_axis_name="s", num_cores=NC, num_subcores=NS
    ),
    scratch_types=[
        pltpu.VMEM_SHARED((PAD_N, H), jnp.float32),
        pltpu.VMEM((C,), jnp.int32),
        pltpu.VMEM((C, H), jnp.float32),
    ],
)


def _dotT(a, w):
    return lax.dot_general(a, w, (((1,), (1,)), ((), ())),
                           preferred_element_type=jnp.float32)


def _lin1_kernel(x_ref, wl_ref, bl_ref, wr_ref, h_ref, xr_ref):
    x = x_ref[...]
    h = _dotT(x, wl_ref[...]) + bl_ref[...]
    h_ref[0] = h[:, :H]
    h_ref[1] = h[:, H:]
    xr_ref[...] = _dotT(x, wr_ref[...])


def _lin1(x, wl, bl, wr):
    return pl.pallas_call(
        _lin1_kernel,
        grid=(N_NODES // RB,),
        in_specs=[
            pl.BlockSpec((RB, D_FEAT), lambda i: (i, 0)),
            pl.BlockSpec((D_FEAT, D_FEAT), lambda i: (0, 0)),
            pl.BlockSpec((1, D_FEAT), lambda i: (0, 0)),
            pl.BlockSpec((D_FEAT, D_FEAT), lambda i: (0, 0)),
        ],
        out_specs=[
            pl.BlockSpec((NC, RB, H), lambda i: (0, i, 0)),
            pl.BlockSpec((RB, D_FEAT), lambda i: (i, 0)),
        ],
        out_shape=[
            jax.ShapeDtypeStruct((NC, N_NODES, H), jnp.float32),
            jax.ShapeDtypeStruct((N_NODES, D_FEAT), jnp.float32),
        ],
    )(x, wl, bl, wr)


def _mid_kernel(sum_ref, cnt_ref, xr_ref, t_ref, st_ref):
    i = pl.program_id(0)
    cnt = cnt_ref[0, :, 0:1] + cnt_ref[1, :, 0:1]
    inv = 1.0 / jnp.maximum(cnt, 1.0)
    agg = jnp.concatenate([sum_ref[0], sum_ref[1]], axis=1) * inv
    t = agg + xr_ref[...]
    t_ref[...] = t

    @pl.when(i == 0)
    def _():
        st_ref[...] = jnp.zeros_like(st_ref)

    st_ref[0:1, :] = st_ref[0:1, :] + jnp.sum(t, axis=0, keepdims=True)
    st_ref[1:2, :] = st_ref[1:2, :] + jnp.sum(t * t, axis=0, keepdims=True)


def _mid(sum_stk, cnt, xr):
    return pl.pallas_call(
        _mid_kernel,
        grid=(N_NODES // RB,),
        in_specs=[
            pl.BlockSpec((NC, RB, H), lambda i: (0, i, 0)),
            pl.BlockSpec((NC, RB, H), lambda i: (0, i, 0)),
            pl.BlockSpec((RB, D_FEAT), lambda i: (i, 0)),
        ],
        out_specs=[
            pl.BlockSpec((RB, D_FEAT), lambda i: (i, 0)),
            pl.BlockSpec((2, D_FEAT), lambda i: (0, 0)),
        ],
        out_shape=[
            jax.ShapeDtypeStruct((N_NODES, D_FEAT), jnp.float32),
            jax.ShapeDtypeStruct((2, D_FEAT), jnp.float32),
        ],
    )(sum_stk, cnt, xr)


def _bn_lin2_kernel(t_ref, st_ref, g_ref, be_ref, wl_ref, bl_ref, wr_ref,
                    h2_ref, x2r_ref):
    n = float(N_NODES)
    mean = st_ref[0:1, :] / n
    var = st_ref[1:2, :] / n - mean * mean
    u = (t_ref[...] - mean) * lax.rsqrt(var + EPS) * g_ref[...] + be_ref[...]
    u = jnp.maximum(u, 0.0)
    h2 = _dotT(u, wl_ref[...]) + bl_ref[...]
    h2_ref[0] = h2[:, :H]
    h2_ref[1] = h2[:, H:]
    x2r_ref[...] = _dotT(u, wr_ref[...])


def _bn_lin2(t, st, gamma, beta, wl, bl, wr):
    return pl.pallas_call(
        _bn_lin2_kernel,
        grid=(N_NODES // RB,),
        in_specs=[
            pl.BlockSpec((RB, D_FEAT), lambda i: (i, 0)),
            pl.BlockSpec((2, D_FEAT), lambda i: (0, 0)),
            pl.BlockSpec((1, D_FEAT), lambda i: (0, 0)),
            pl.BlockSpec((1, D_FEAT), lambda i: (0, 0)),
            pl.BlockSpec((D_FEAT, D_FEAT), lambda i: (0, 0)),
            pl.BlockSpec((1, D_FEAT), lambda i: (0, 0)),
            pl.BlockSpec((D_FEAT, D_FEAT), lambda i: (0, 0)),
        ],
        out_specs=[
            pl.BlockSpec((NC, RB, H), lambda i: (0, i, 0)),
            pl.BlockSpec((RB, D_FEAT), lambda i: (i, 0)),
        ],
        out_shape=[
            jax.ShapeDtypeStruct((NC, N_NODES, H), jnp.float32),
            jax.ShapeDtypeStruct((N_NODES, D_FEAT), jnp.float32),
        ],
    )(t, st, gamma, beta, wl, bl, wr)


def _out_kernel(sum_ref, cnt_ref, x2r_ref, o_ref):
    cnt = cnt_ref[0, :, 0:1] + cnt_ref[1, :, 0:1]
    inv = 1.0 / jnp.maximum(cnt, 1.0)
    agg = jnp.concatenate([sum_ref[0], sum_ref[1]], axis=1) * inv
    o_ref[...] = agg + x2r_ref[...]


def _out(sum_stk, cnt, x2r):
    return pl.pallas_call(
        _out_kernel,
        grid=(N_NODES // RB,),
        in_specs=[
            pl.BlockSpec((NC, RB, H), lambda i: (0, i, 0)),
            pl.BlockSpec((NC, RB, H), lambda i: (0, i, 0)),
            pl.BlockSpec((RB, D_FEAT), lambda i: (i, 0)),
        ],
        out_specs=pl.BlockSpec((RB, D_FEAT), lambda i: (i, 0)),
        out_shape=jax.ShapeDtypeStruct((N_NODES, D_FEAT), jnp.float32),
    )(sum_stk, cnt, x2r)


@jax.jit
def kernel(x, edge_index, W1l, b1l, W1r, bn_gamma, bn_beta, W2l, b2l, W2r):
    npad = PAD_E - N_EDGES
    src = jnp.concatenate(
        [edge_index[0].astype(jnp.int32), jnp.zeros((npad,), jnp.int32)])
    dst = jnp.concatenate(
        [edge_index[1].astype(jnp.int32),
         jnp.full((npad,), N_NODES, jnp.int32)])
    bsrc = jnp.concatenate([src, src + N_NODES])
    ones_in = jnp.ones((C, H), jnp.float32)
    zeros_in = jnp.zeros((C, H), jnp.float32)

    cnt = _sc_cnt(dst, ones_in, zeros_in).reshape(NC, PAD_N, H)
    h_stk, xr = _lin1(x, W1l, b1l.reshape(1, -1), W1r)
    sum_stk = _sc_agg(h_stk.reshape(NC * N_NODES, H), bsrc, dst, zeros_in)
    t, st = _mid(sum_stk.reshape(NC, PAD_N, H), cnt, xr)
    h2_stk, x2r = _bn_lin2(t, st, bn_gamma.reshape(1, -1), bn_beta.reshape(1, -1),
                           W2l, b2l.reshape(1, -1), W2r)
    sum2_stk = _sc_agg(h2_stk.reshape(NC * N_NODES, H), bsrc, dst, zeros_in)
    return _out(sum2_stk.reshape(NC, PAD_N, H), cnt, x2r)


# speedup vs baseline: 3.5942x; 1.0796x over previous
import functools

import jax
import jax.numpy as jnp
from jax import lax
from jax.experimental import pallas as pl
from jax.experimental.pallas import tpu as pltpu
from jax.experimental.pallas import tpu_sc as plsc

N_NODES = 10000
N_EDGES = 160000
D_FEAT = 256
H = 128
NC, NS = 2, 16
C = 128
PAD_E = 163840
EPT = PAD_E // NS
NCH = EPT // C
CCH = PAD_E // (NC * NS * C)
PAD_N = 10240
RPT = PAD_N // NS
RW = 128
NWB = RPT // RW
RB = 2000
EPS = 1e-5


def _sc_agg_kernel(h_flat, bsrc_hbm, dst_hbm, zeros_hbm, sum_flat,
                   acc, sv, da0, da1, rv0, rv1, sem0, sem1, semd0, semd1):
    c = lax.axis_index("c")
    s = lax.axis_index("s")
    base = s * RPT

    pltpu.sync_copy(zeros_hbm, rv0)
    for k in range(NWB):
        pltpu.sync_copy(rv0, acc.at[pl.ds(base + k * RW, RW)])

    pltpu.sync_copy(bsrc_hbm.at[pl.ds(c * PAD_E + s * EPT, EPT)], sv)
    plsc.subcore_barrier()

    def dst_start(j, da, semd):
        return pltpu.async_copy(dst_hbm.at[pl.ds((s * NCH + j) * C, C)], da, semd)

    def dst_wait(j, da, semd):
        pltpu.make_async_copy(dst_hbm.at[pl.ds((s * NCH + j) * C, C)], da,
                              semd).wait()

    def g_start(j, rv, sem):
        return pltpu.async_copy(h_flat.at[sv.at[pl.ds(j * C, C)]], rv, sem)

    def g_wait(j, rv, sem):
        pltpu.make_async_copy(h_flat.at[sv.at[pl.ds(j * C, C)]], rv, sem).wait()

    dst_start(0, da0, semd0)
    g_start(0, rv0, sem0)

    def pair(p, carry):
        j0 = 2 * p
        j1 = 2 * p + 1
        dst_start(j1, da1, semd1)
        g_start(j1, rv1, sem1)
        dst_wait(j0, da0, semd0)
        g_wait(j0, rv0, sem0)
        pltpu.sync_copy(rv0, acc.at[da0], add=True)

        @pl.when(p < NCH // 2 - 1)
        def _():
            dst_start(j0 + 2, da0, semd0)
            g_start(j0 + 2, rv0, sem0)

        dst_wait(j1, da1, semd1)
        g_wait(j1, rv1, sem1)
        pltpu.sync_copy(rv1, acc.at[da1], add=True)
        return carry

    lax.fori_loop(0, NCH // 2, pair, 0)
    plsc.subcore_barrier()

    for k in range(NWB):
        b = base + k * RW
        pltpu.sync_copy(acc.at[pl.ds(b, RW)],
                        sum_flat.at[pl.ds(c * PAD_N + b, RW)])


_sc_agg = pl.kernel(
    _sc_agg_kernel,
    out_type=jax.ShapeDtypeStruct((NC * PAD_N, H), jnp.float32),
    mesh=plsc.VectorSubcoreMesh(
        core_axis_name="c", subcore_axis_name="s", num_cores=NC, num_subcores=NS
    ),
    scratch_types=[
        pltpu.VMEM_SHARED((PAD_N, H), jnp.float32),
        pltpu.VMEM((EPT,), jnp.int32),
        pltpu.VMEM((C,), jnp.int32),
        pltpu.VMEM((C,), jnp.int32),
        pltpu.VMEM((C, H), jnp.float32),
        pltpu.VMEM((C, H), jnp.float32),
        pltpu.SemaphoreType.DMA,
        pltpu.SemaphoreType.DMA,
        pltpu.SemaphoreType.DMA,
        pltpu.SemaphoreType.DMA,
    ],
)


def _sc_cnt_kernel(dst_hbm, ones_hbm, zeros_hbm, cnt_out, cacc, dst_c, ones_v):
    c = lax.axis_index("c")
    s = lax.axis_index("s")
    w = c * NS + s
    base = s * RPT

    pltpu.sync_copy(zeros_hbm, ones_v)
    for k in range(NWB):
        pltpu.sync_copy(ones_v, cacc.at[pl.ds(base + k * RW, RW)])
    pltpu.sync_copy(ones_hbm, ones_v)
    plsc.subcore_barrier()

    def chunk(j, carry):
        pltpu.sync_copy(dst_hbm.at[pl.ds((w * CCH + j) * C, C)], dst_c)
        pltpu.sync_copy(ones_v, cacc.at[dst_c], add=True)
        return carry

    lax.fori_loop(0, CCH, chunk, 0)
    plsc.subcore_barrier()

    for k in range(NWB):
        b = base + k * RW
        pltpu.sync_copy(cacc.at[pl.ds(b, RW)],
                        cnt_out.at[pl.ds(c * PAD_N + b, RW)])


_sc_cnt = pl.kernel(
    _sc_cnt_kernel,
    out_type=jax.ShapeDtypeStruct((NC * PAD_N, H), jnp.float32),
    mesh=plsc.VectorSubcoreMesh(
        core_axis_name="c", subcore_axis_name="s", num_cores=NC, num_subcores=NS
    ),
    scratch_types=[
        pltpu.VMEM_SHARED((PAD_N, H), jnp.float32),
        pltpu.VMEM((C,), jnp.int32),
        pltpu.VMEM((C, H), jnp.float32),
    ],
)


def _dotT(a, w):
    return lax.dot_general(a, w, (((1,), (1,)), ((), ())),
                           preferred_element_type=jnp.float32)


def _lin1_kernel(x_ref, wl_ref, bl_ref, wr_ref, h_ref, xr_ref):
    x = x_ref[...]
    h = _dotT(x, wl_ref[...]) + bl_ref[...]
    h_ref[0] = h[:, :H]
    h_ref[1] = h[:, H:]
    xr_ref[...] = _dotT(x, wr_ref[...])


def _lin1(x, wl, bl, wr):
    return pl.pallas_call(
        _lin1_kernel,
        grid=(N_NODES // RB,),
        in_specs=[
            pl.BlockSpec((RB, D_FEAT), lambda i: (i, 0)),
            pl.BlockSpec((D_FEAT, D_FEAT), lambda i: (0, 0)),
            pl.BlockSpec((1, D_FEAT), lambda i: (0, 0)),
            pl.BlockSpec((D_FEAT, D_FEAT), lambda i: (0, 0)),
        ],
        out_specs=[
            pl.BlockSpec((NC, RB, H), lambda i: (0, i, 0)),
            pl.BlockSpec((RB, D_FEAT), lambda i: (i, 0)),
        ],
        out_shape=[
            jax.ShapeDtypeStruct((NC, N_NODES, H), jnp.float32),
            jax.ShapeDtypeStruct((N_NODES, D_FEAT), jnp.float32),
        ],
    )(x, wl, bl, wr)


def _mid_kernel(sum_ref, cnt_ref, xr_ref, t_ref, st_ref):
    i = pl.program_id(0)
    cnt = cnt_ref[0, :, 0:1] + cnt_ref[1, :, 0:1]
    inv = 1.0 / jnp.maximum(cnt, 1.0)
    agg = jnp.concatenate([sum_ref[0], sum_ref[1]], axis=1) * inv
    t = agg + xr_ref[...]
    t_ref[...] = t

    @pl.when(i == 0)
    def _():
        st_ref[...] = jnp.zeros_like(st_ref)

    st_ref[0:1, :] = st_ref[0:1, :] + jnp.sum(t, axis=0, keepdims=True)
    st_ref[1:2, :] = st_ref[1:2, :] + jnp.sum(t * t, axis=0, keepdims=True)


def _mid(sum_stk, cnt, xr):
    return pl.pallas_call(
        _mid_kernel,
        grid=(N_NODES // RB,),
        in_specs=[
            pl.BlockSpec((NC, RB, H), lambda i: (0, i, 0)),
            pl.BlockSpec((NC, RB, H), lambda i: (0, i, 0)),
            pl.BlockSpec((RB, D_FEAT), lambda i: (i, 0)),
        ],
        out_specs=[
            pl.BlockSpec((RB, D_FEAT), lambda i: (i, 0)),
            pl.BlockSpec((2, D_FEAT), lambda i: (0, 0)),
        ],
        out_shape=[
            jax.ShapeDtypeStruct((N_NODES, D_FEAT), jnp.float32),
            jax.ShapeDtypeStruct((2, D_FEAT), jnp.float32),
        ],
    )(sum_stk, cnt, xr)


def _bn_lin2_kernel(t_ref, st_ref, g_ref, be_ref, wl_ref, bl_ref, wr_ref,
                    h2_ref, x2r_ref):
    n = float(N_NODES)
    mean = st_ref[0:1, :] / n
    var = st_ref[1:2, :] / n - mean * mean
    u = (t_ref[...] - mean) * lax.rsqrt(var + EPS) * g_ref[...] + be_ref[...]
    u = jnp.maximum(u, 0.0)
    h2 = _dotT(u, wl_ref[...]) + bl_ref[...]
    h2_ref[0] = h2[:, :H]
    h2_ref[1] = h2[:, H:]
    x2r_ref[...] = _dotT(u, wr_ref[...])


def _bn_lin2(t, st, gamma, beta, wl, bl, wr):
    return pl.pallas_call(
        _bn_lin2_kernel,
        grid=(N_NODES // RB,),
        in_specs=[
            pl.BlockSpec((RB, D_FEAT), lambda i: (i, 0)),
            pl.BlockSpec((2, D_FEAT), lambda i: (0, 0)),
            pl.BlockSpec((1, D_FEAT), lambda i: (0, 0)),
            pl.BlockSpec((1, D_FEAT), lambda i: (0, 0)),
            pl.BlockSpec((D_FEAT, D_FEAT), lambda i: (0, 0)),
            pl.BlockSpec((1, D_FEAT), lambda i: (0, 0)),
            pl.BlockSpec((D_FEAT, D_FEAT), lambda i: (0, 0)),
        ],
        out_specs=[
            pl.BlockSpec((NC, RB, H), lambda i: (0, i, 0)),
            pl.BlockSpec((RB, D_FEAT), lambda i: (i, 0)),
        ],
        out_shape=[
            jax.ShapeDtypeStruct((NC, N_NODES, H), jnp.float32),
            jax.ShapeDtypeStruct((N_NODES, D_FEAT), jnp.float32),
        ],
    )(t, st, gamma, beta, wl, bl, wr)


def _out_kernel(sum_ref, cnt_ref, x2r_ref, o_ref):
    cnt = cnt_ref[0, :, 0:1] + cnt_ref[1, :, 0:1]
    inv = 1.0 / jnp.maximum(cnt, 1.0)
    agg = jnp.concatenate([sum_ref[0], sum_ref[1]], axis=1) * inv
    o_ref[...] = agg + x2r_ref[...]


def _out(sum_stk, cnt, x2r):
    return pl.pallas_call(
        _out_kernel,
        grid=(N_NODES // RB,),
        in_specs=[
            pl.BlockSpec((NC, RB, H), lambda i: (0, i, 0)),
            pl.BlockSpec((NC, RB, H), lambda i: (0, i, 0)),
            pl.BlockSpec((RB, D_FEAT), lambda i: (i, 0)),
        ],
        out_specs=pl.BlockSpec((RB, D_FEAT), lambda i: (i, 0)),
        out_shape=jax.ShapeDtypeStruct((N_NODES, D_FEAT), jnp.float32),
    )(sum_stk, cnt, x2r)


@jax.jit
def kernel(x, edge_index, W1l, b1l, W1r, bn_gamma, bn_beta, W2l, b2l, W2r):
    npad = PAD_E - N_EDGES
    src = jnp.concatenate(
        [edge_index[0].astype(jnp.int32), jnp.zeros((npad,), jnp.int32)])
    dst = jnp.concatenate(
        [edge_index[1].astype(jnp.int32),
         jnp.full((npad,), N_NODES, jnp.int32)])
    bsrc = jnp.concatenate([src, src + N_NODES])
    ones_in = jnp.ones((C, H), jnp.float32)
    zeros_in = jnp.zeros((C, H), jnp.float32)

    cnt = _sc_cnt(dst, ones_in, zeros_in).reshape(NC, PAD_N, H)
    h_stk, xr = _lin1(x, W1l, b1l.reshape(1, -1), W1r)
    sum_stk = _sc_agg(h_stk.reshape(NC * N_NODES, H), bsrc, dst, zeros_in)
    t, st = _mid(sum_stk.reshape(NC, PAD_N, H), cnt, xr)
    h2_stk, x2r = _bn_lin2(t, st, bn_gamma.reshape(1, -1), bn_beta.reshape(1, -1),
                           W2l, b2l.reshape(1, -1), W2r)
    sum2_stk = _sc_agg(h2_stk.reshape(NC * N_NODES, H), bsrc, dst, zeros_in)
    return _out(sum2_stk.reshape(NC, PAD_N, H), cnt, x2r)
